# baseline (speedup 1.0000x reference)
import jax
import jax.numpy as jnp
from jax import lax
from jax.experimental import pallas as pl
from jax.experimental.pallas import tpu as pltpu
from jax.experimental.pallas import tpu_sc as plsc

MAX_LEN = 200
EMB_DIM = 32
BATCH = 4096
N_IDX = BATCH * MAX_LEN
N_GRP = MAX_LEN // 4
N_ROWS = N_IDX // 4

_info = plsc.get_sparse_core_info()
NC, NS = _info.num_cores, _info.num_subcores
NW = NC * NS
PER_W = N_IDX // NW
CHUNK = 1024
N_CHUNKS = PER_W // CHUNK


def _gather_body(x_hbm, table_hbm, out_hbm, idx_v, rows_v, sem):
    wid = lax.axis_index("s") * NC + lax.axis_index("c")
    base = wid * PER_W

    def chunk_body(i, carry):
        off = base + i * CHUNK
        pltpu.sync_copy(x_hbm.at[pl.ds(off, CHUNK)], idx_v)
        pltpu.async_copy(table_hbm.at[idx_v], rows_v, sem).wait()
        pltpu.sync_copy(rows_v, out_hbm.at[pl.ds(off, CHUNK)])
        return carry

    lax.fori_loop(0, N_CHUNKS, chunk_body, 0)


def _sc_gather(x_km, table):
    mesh = plsc.VectorSubcoreMesh(core_axis_name="c", subcore_axis_name="s")
    kern = pl.kernel(
        _gather_body,
        mesh=mesh,
        out_type=jax.ShapeDtypeStruct((N_IDX, EMB_DIM), jnp.float32),
        scratch_types=[
            pltpu.VMEM((CHUNK,), jnp.int32),
            pltpu.VMEM((CHUNK, EMB_DIM), jnp.float32),
            pltpu.SemaphoreType.DMA,
        ],
        compiler_params=pltpu.CompilerParams(use_tc_tiling_on_sc=False),
    )
    return kern(x_km, table)


BB = 512
NB = BATCH // BB


def _mlp_body(emb_ref, w1_ref, b1_ref, w2_ref, b2_ref, out_ref, acc_ref):
    k = pl.program_id(1)

    @pl.when(k == 0)
    def _():
        acc_ref[...] = jnp.zeros_like(acc_ref)

    acc_ref[...] += jnp.dot(
        emb_ref[...], w1_ref[...], preferred_element_type=jnp.float32
    )

    @pl.when(k == N_GRP - 1)
    def _():
        h = jnp.maximum(acc_ref[...] + b1_ref[...], 0.0)
        o = jnp.dot(h, w2_ref[...], preferred_element_type=jnp.float32)
        out_ref[...] = jax.nn.sigmoid(o + b2_ref[...])


def _tc_mlp(emb128, w1t, b1, w2t, b2):
    f = pl.pallas_call(
        _mlp_body,
        grid=(NB, N_GRP),
        in_specs=[
            pl.BlockSpec((BB, 128), lambda i, k: (k * NB + i, 0)),
            pl.BlockSpec((128, 32), lambda i, k: (k, 0)),
            pl.BlockSpec((1, 32), lambda i, k: (0, 0)),
            pl.BlockSpec((32, 1), lambda i, k: (0, 0)),
            pl.BlockSpec((1, 1), lambda i, k: (0, 0)),
        ],
        out_specs=pl.BlockSpec((BB, 1), lambda i, k: (i, 0)),
        out_shape=jax.ShapeDtypeStruct((BATCH, 1), jnp.float32),
        scratch_shapes=[pltpu.VMEM((BB, 32), jnp.float32)],
    )
    return f(emb128, w1t, b1, w2t, b2)


@jax.jit
def kernel(x, table, W1, b1, W2, b2):
    x_km = x.reshape(BATCH, N_GRP, 4).transpose(1, 0, 2).reshape(-1)
    emb128 = _sc_gather(x_km, table).reshape(N_ROWS, 128)
    return _tc_mlp(emb128, W1.T, b1.reshape(1, 32), W2.T, b2.reshape(1, 1))
